# baseline (speedup 1.0000x reference)
import functools

import jax
import jax.numpy as jnp
from jax import lax
from jax.experimental import pallas as pl
from jax.experimental.pallas import tpu as pltpu
from jax.experimental.pallas import tpu_sc as plsc

NC, NS, L = 2, 16, 16
NW = NC * NS
TB = 128
NBK = 48
NP = NBK * TB
RPW = NP // NW
GC = 48
CH = 32
SQRT1_2 = 0.7071067811865476


def _route_body(ctx_ref, x_ref, W1_ref, b1_ref, W2_ref, b2_ref, WgT_ref,
                bg_ref, e1_ref, e2_ref, r1_ref, r2_ref, w1_ref, w2_ref,
                base_ref, be_ref, carry_ref):
    i = pl.program_id(0)
    nb = pl.num_programs(0)
    ts, m = e1_ref.shape
    h = jnp.dot(ctx_ref[...], W1_ref[...],
                preferred_element_type=jnp.float32) + b1_ref[...]
    h = 0.5 * h * (1.0 + lax.erf(h * SQRT1_2))
    logits = jnp.dot(h, W2_ref[...],
                     preferred_element_type=jnp.float32) + b2_ref[...]
    mx = jnp.max(logits, axis=-1, keepdims=True)
    ex = jnp.exp(logits - mx)
    p = ex / jnp.sum(ex, axis=-1, keepdims=True)
    col = lax.broadcasted_iota(jnp.int32, p.shape, 1)
    big = jnp.int32(0x7FFFFFFF)
    m1 = jnp.max(p, axis=-1, keepdims=True)
    a1 = jnp.min(jnp.where(p == m1, col, big), axis=-1, keepdims=True)
    pm = jnp.where(col == a1, -1.0, p)
    m2 = jnp.max(pm, axis=-1, keepdims=True)
    a2 = jnp.min(jnp.where(pm == m2, col, big), axis=-1, keepdims=True)
    mask = (col == a1) | (col == a2)
    s = jnp.where(mask, p, 0.0)
    s = s / jnp.sum(s, axis=-1, keepdims=True)
    timing = jax.nn.sigmoid(
        jnp.dot(x_ref[...], WgT_ref[...],
                preferred_element_type=jnp.float32) + bg_ref[...])
    w = s * timing

    @pl.when(i == 0)
    def _init():
        carry_ref[...] = jnp.zeros_like(carry_ref)

    maskf = mask.astype(jnp.float32)
    rt = lax.broadcasted_iota(jnp.int32, (ts, ts), 0)
    ct = lax.broadcasted_iota(jnp.int32, (ts, ts), 1)
    strict_l = (rt > ct).astype(jnp.float32)
    rank = carry_ref[...] + jnp.dot(strict_l, maskf,
                                    preferred_element_type=jnp.float32)
    carry_ref[...] += jnp.sum(maskf, axis=0, keepdims=True)
    ranki = rank.astype(jnp.int32)

    e1_ref[...] = jnp.broadcast_to(a1, (ts, m))
    e2_ref[...] = jnp.broadcast_to(a2, (ts, m))
    r1_ref[...] = jnp.broadcast_to(
        jnp.sum(jnp.where(col == a1, ranki, 0), axis=-1, keepdims=True),
        (ts, m))
    r2_ref[...] = jnp.broadcast_to(
        jnp.sum(jnp.where(col == a2, ranki, 0), axis=-1, keepdims=True),
        (ts, m))
    w1_ref[...] = jnp.broadcast_to(
        jnp.sum(jnp.where(col == a1, w, 0.0), axis=-1, keepdims=True),
        (ts, m))
    w2_ref[...] = jnp.broadcast_to(
        jnp.sum(jnp.where(col == a2, w, 0.0), axis=-1, keepdims=True),
        (ts, m))

    @pl.when(i == nb - 1)
    def _fin():
        cnt = carry_ref[...]
        pb = jnp.ceil(cnt / TB)
        mi = lax.broadcasted_iota(jnp.int32, (m, m), 0)
        mj = lax.broadcasted_iota(jnp.int32, (m, m), 1)
        sl16 = (mi < mj).astype(jnp.float32)
        bb = jnp.dot(pb, sl16, preferred_element_type=jnp.float32)
        base_ref[...] = (bb * TB).astype(jnp.int32)
        eye = mi == mj
        bbc = jnp.sum(jnp.where(eye, jnp.broadcast_to(bb, (m, m)), 0.0),
                      axis=1, keepdims=True)
        nbw = be_ref.shape[1]
        bidx = lax.broadcasted_iota(jnp.int32, (m, nbw), 1)
        bbci = bbc.astype(jnp.int32)
        geq = (jnp.broadcast_to(bbci, (m, nbw)) <= bidx).astype(jnp.int32)
        be_ref[...] = jnp.sum(geq, axis=0, keepdims=True) - 1


def _sc_build_gather(x_hbm, e1_hbm, e2_hbm, r1_hbm, r2_hbm, w1_hbm, w2_hbm,
                     base_hbm, xg_hbm, roww_hbm, e1v, e2v, r1v, r2v, w1v, w2v,
                     basev, loc_tok, loc_w, idxv0, idxv1, rowsv0, rowsv1,
                     sem0, sem1):
    idxv = (idxv0, idxv1)
    rowsv = (rowsv0, rowsv1)
    sem = (sem0, sem1)
    s = e1v.shape[0]
    wid = lax.axis_index("s") * NC + lax.axis_index("c")
    rb = wid * RPW
    pltpu.sync_copy(e1_hbm, e1v)
    pltpu.sync_copy(e2_hbm, e2v)
    pltpu.sync_copy(r1_hbm, r1v)
    pltpu.sync_copy(r2_hbm, r2v)
    pltpu.sync_copy(w1_hbm, w1v)
    pltpu.sync_copy(w2_hbm, w2v)
    pltpu.sync_copy(base_hbm, basev)
    for t in range(RPW // L):
        loc_tok[pl.ds(t * L, L)] = jnp.zeros((L,), jnp.int32)
        loc_w[pl.ds(t * L, L)] = jnp.zeros((L,), jnp.float32)

    def scan_body(j, _):
        tok = j * L + lax.iota(jnp.int32, L)
        for ev, rv, wv in ((e1v, r1v, w1v), (e2v, r2v, w2v)):
            e = ev[pl.ds(j * L, L)]
            r = rv[pl.ds(j * L, L)]
            wgt = wv[pl.ds(j * L, L)]
            pos = plsc.load_gather(basev, [e]) + r
            msk = (pos >= rb) & (pos < rb + RPW)
            il = jnp.clip(pos - rb, 0, RPW - 1)
            plsc.store_scatter(loc_tok, [il], tok, mask=msk)
            plsc.store_scatter(loc_w, [il], wgt, mask=msk)
        return 0

    lax.fori_loop(0, s // L, scan_body, 0)
    pltpu.sync_copy(loc_w, roww_hbm.at[pl.ds(rb, RPW)])
    nch = RPW // GC
    for t in range(GC // L):
        idxv[0][pl.ds(t * L, L)] = loc_tok[pl.ds(t * L, L)]
    cps = [None] * nch
    cps[0] = pltpu.async_copy(x_hbm.at[idxv[0]], rowsv[0], sem[0])
    for cchunk in range(nch):
        if cchunk + 1 < nch:
            nb_ = (cchunk + 1) % 2
            for t in range(GC // L):
                idxv[nb_][pl.ds(t * L, L)] = loc_tok[
                    pl.ds((cchunk + 1) * GC + t * L, L)]
            cps[cchunk + 1] = pltpu.async_copy(
                x_hbm.at[idxv[nb_]], rowsv[nb_], sem[nb_])
        cps[cchunk].wait()
        pltpu.sync_copy(rowsv[cchunk % 2],
                        xg_hbm.at[pl.ds(rb + cchunk * GC, GC)])


def _group_mm_body(be_ref, xg_ref, Wt_ref, bt_ref, ch_ref, w_ref, y_ref):
    y = jnp.dot(xg_ref[...].astype(jnp.bfloat16), Wt_ref[0],
                preferred_element_type=jnp.float32)
    y_ref[...] = (y + bt_ref[0] + ch_ref[0]) * w_ref[...]


def _sc_combine(y_hbm, e1_hbm, e2_hbm, r1_hbm, r2_hbm, base_hbm, sel_hbm,
                e1v, e2v, r1v, r2v, basev, p1v, p2v, p1c, p2c,
                buf_a, buf_b, sem):
    tpw = p1v.shape[0]
    d = buf_a.shape[1]
    wid = lax.axis_index("s") * NC + lax.axis_index("c")
    tb = wid * tpw
    pltpu.sync_copy(e1_hbm.at[pl.ds(tb, tpw)], e1v)
    pltpu.sync_copy(e2_hbm.at[pl.ds(tb, tpw)], e2v)
    pltpu.sync_copy(r1_hbm.at[pl.ds(tb, tpw)], r1v)
    pltpu.sync_copy(r2_hbm.at[pl.ds(tb, tpw)], r2v)
    pltpu.sync_copy(base_hbm, basev)
    for t in range(tpw // L):
        sl = pl.ds(t * L, L)
        p1v[sl] = plsc.load_gather(basev, [e1v[sl]]) + r1v[sl]
        p2v[sl] = plsc.load_gather(basev, [e2v[sl]]) + r2v[sl]
    for cchunk in range(tpw // CH):
        for t in range(CH // L):
            p1c[pl.ds(t * L, L)] = p1v[pl.ds(cchunk * CH + t * L, L)]
            p2c[pl.ds(t * L, L)] = p2v[pl.ds(cchunk * CH + t * L, L)]
        pltpu.async_copy(y_hbm.at[p1c], buf_a, sem).wait()
        pltpu.async_copy(y_hbm.at[p2c], buf_b, sem).wait()

        def add_row(rr, _):
            for cc in range(d // L):
                sl = pl.ds(cc * L, L)
                buf_a[rr, sl] = buf_a[rr, sl] + buf_b[rr, sl]
            return 0

        lax.fori_loop(0, CH, add_row, 0)
        pltpu.sync_copy(buf_a, sel_hbm.at[pl.ds(tb + cchunk * CH, CH)])


def _sc_mesh():
    return plsc.VectorSubcoreMesh(core_axis_name="c", subcore_axis_name="s",
                                  num_cores=NC, num_subcores=NS)


def _run_build_gather(xs, e1, e2, r1, r2, w1, w2, base16):
    S, D = xs.shape
    M = base16.shape[0]
    build = functools.partial(
        pl.kernel,
        out_type=[
            jax.ShapeDtypeStruct((NP, D), jnp.float32),
            jax.ShapeDtypeStruct((NP,), jnp.float32),
        ],
        mesh=_sc_mesh(),
        compiler_params=pltpu.CompilerParams(needs_layout_passes=False),
        scratch_types=[
            pltpu.VMEM((S,), jnp.int32),
            pltpu.VMEM((S,), jnp.int32),
            pltpu.VMEM((S,), jnp.int32),
            pltpu.VMEM((S,), jnp.int32),
            pltpu.VMEM((S,), jnp.float32),
            pltpu.VMEM((S,), jnp.float32),
            pltpu.VMEM((M,), jnp.int32),
            pltpu.VMEM((RPW,), jnp.int32),
            pltpu.VMEM((RPW,), jnp.float32),
            pltpu.VMEM((GC,), jnp.int32),
            pltpu.VMEM((GC,), jnp.int32),
            pltpu.VMEM((GC, D), jnp.float32),
            pltpu.VMEM((GC, D), jnp.float32),
            pltpu.SemaphoreType.DMA,
            pltpu.SemaphoreType.DMA,
        ],
    )(_sc_build_gather)
    return build(xs, e1, e2, r1, r2, w1, w2, base16)


def _run_combine(y, e1, e2, r1, r2, base16):
    S = e1.shape[0]
    D = y.shape[1]
    M = base16.shape[0]
    tpw = S // NW
    combine = functools.partial(
        pl.kernel,
        out_type=jax.ShapeDtypeStruct((S, D), jnp.float32),
        mesh=_sc_mesh(),
        compiler_params=pltpu.CompilerParams(needs_layout_passes=False),
        scratch_types=[
            pltpu.VMEM((tpw,), jnp.int32),
            pltpu.VMEM((tpw,), jnp.int32),
            pltpu.VMEM((tpw,), jnp.int32),
            pltpu.VMEM((tpw,), jnp.int32),
            pltpu.VMEM((M,), jnp.int32),
            pltpu.VMEM((tpw,), jnp.int32),
            pltpu.VMEM((tpw,), jnp.int32),
            pltpu.VMEM((CH,), jnp.int32),
            pltpu.VMEM((CH,), jnp.int32),
            pltpu.VMEM((CH, D), jnp.float32),
            pltpu.VMEM((CH, D), jnp.float32),
            pltpu.SemaphoreType.DMA,
        ],
    )(_sc_combine)
    return combine(y, e1, e2, r1, r2, base16)


def _integrate_body(x_ref, sel_ref, Wi_ref, bi_ref, out_ref):
    d = x_ref.shape[-1]
    out_ref[...] = (
        jnp.dot(x_ref[...], Wi_ref[:d, :], preferred_element_type=jnp.float32)
        + jnp.dot(sel_ref[...], Wi_ref[d:, :],
                  preferred_element_type=jnp.float32)
        + bi_ref[...])


def kernel(x, context, Wt, bt, Wg, bg, ch, W1, b1, W2, b2, Wi, bi):
    B, S, D = x.shape
    M = Wt.shape[0]
    H = W1.shape[1]
    xs = x.reshape(S, D)
    cs = context.reshape(S, D)
    TS = 256
    n_blk = S // TS

    outs = pl.pallas_call(
        _route_body,
        grid=(n_blk,),
        in_specs=[
            pl.BlockSpec((TS, D), lambda i: (i, 0)),
            pl.BlockSpec((TS, D), lambda i: (i, 0)),
            pl.BlockSpec((D, H), lambda i: (0, 0)),
            pl.BlockSpec((1, H), lambda i: (0, 0)),
            pl.BlockSpec((H, M), lambda i: (0, 0)),
            pl.BlockSpec((1, M), lambda i: (0, 0)),
            pl.BlockSpec((D, M), lambda i: (0, 0)),
            pl.BlockSpec((1, M), lambda i: (0, 0)),
        ],
        out_specs=[
            pl.BlockSpec((TS, M), lambda i: (i, 0)),
            pl.BlockSpec((TS, M), lambda i: (i, 0)),
            pl.BlockSpec((TS, M), lambda i: (i, 0)),
            pl.BlockSpec((TS, M), lambda i: (i, 0)),
            pl.BlockSpec((TS, M), lambda i: (i, 0)),
            pl.BlockSpec((TS, M), lambda i: (i, 0)),
            pl.BlockSpec((1, M), lambda i: (0, 0)),
            pl.BlockSpec((1, 64), lambda i: (0, 0)),
        ],
        out_shape=[
            jax.ShapeDtypeStruct((S, M), jnp.int32),
            jax.ShapeDtypeStruct((S, M), jnp.int32),
            jax.ShapeDtypeStruct((S, M), jnp.int32),
            jax.ShapeDtypeStruct((S, M), jnp.int32),
            jax.ShapeDtypeStruct((S, M), jnp.float32),
            jax.ShapeDtypeStruct((S, M), jnp.float32),
            jax.ShapeDtypeStruct((1, M), jnp.int32),
            jax.ShapeDtypeStruct((1, 64), jnp.int32),
        ],
        scratch_shapes=[pltpu.VMEM((1, M), jnp.float32)],
        compiler_params=pltpu.CompilerParams(
            dimension_semantics=("arbitrary",)),
    )(cs, xs, W1, b1.reshape(1, H), W2, b2.reshape(1, M), Wg.T,
      bg.reshape(1, M))
    e1b, e2b, r1b, r2b, w1b, w2b, base, beb = outs
    e1, e2, r1, r2 = e1b[:, 0], e2b[:, 0], r1b[:, 0], r2b[:, 0]
    w1, w2 = w1b[:, 0], w2b[:, 0]
    base16 = base.reshape(M)
    be = beb.reshape(64)[:NBK]

    xg, roww = _run_build_gather(xs, e1, e2, r1, r2, w1, w2, base16)
    y = pl.pallas_call(
        _group_mm_body,
        grid_spec=pltpu.PrefetchScalarGridSpec(
            num_scalar_prefetch=1,
            grid=(NBK,),
            in_specs=[
                pl.BlockSpec((TB, D), lambda i, be: (i, 0)),
                pl.BlockSpec((1, D, D), lambda i, be: (be[i], 0, 0)),
                pl.BlockSpec((1, 1, D), lambda i, be: (be[i], 0, 0)),
                pl.BlockSpec((1, 1, D), lambda i, be: (be[i], 0, 0)),
                pl.BlockSpec((TB, 1), lambda i, be: (i, 0)),
            ],
            out_specs=pl.BlockSpec((TB, D), lambda i, be: (i, 0)),
        ),
        out_shape=jax.ShapeDtypeStruct((NP, D), jnp.float32),
        compiler_params=pltpu.CompilerParams(
            dimension_semantics=("arbitrary",)),
    )(be, xg, Wt.astype(jnp.bfloat16), bt.reshape(M, 1, D), ch.reshape(M, 1, D), roww.reshape(NP, 1))

    sel = _run_combine(y, e1, e2, r1, r2, base16)

    out = pl.pallas_call(
        _integrate_body,
        grid=(n_blk,),
        in_specs=[
            pl.BlockSpec((TS, D), lambda i: (i, 0)),
            pl.BlockSpec((TS, D), lambda i: (i, 0)),
            pl.BlockSpec((2 * D, D), lambda i: (0, 0)),
            pl.BlockSpec((1, D), lambda i: (0, 0)),
        ],
        out_specs=pl.BlockSpec((TS, D), lambda i: (i, 0)),
        out_shape=jax.ShapeDtypeStruct((S, D), jnp.float32),
    )(xs, sel, Wi, bi.reshape(1, D))

    return out.reshape(B, S, D)


# speedup vs baseline: 1.0034x; 1.0034x over previous
import functools

import jax
import jax.numpy as jnp
from jax import lax
from jax.experimental import pallas as pl
from jax.experimental.pallas import tpu as pltpu
from jax.experimental.pallas import tpu_sc as plsc

NC, NS, L = 2, 16, 16
NW = NC * NS
TB = 128
NBK = 48
NP = NBK * TB
RPW = NP // NW
GC = 48
NBUF = 7
NOUT = 6
CH = 32
SQRT1_2 = 0.7071067811865476


def _route_body(ctx_ref, x_ref, W1_ref, b1_ref, W2_ref, b2_ref, WgT_ref,
                bg_ref, e1_ref, e2_ref, r1_ref, r2_ref, w1_ref, w2_ref,
                base_ref, be_ref, carry_ref):
    i = pl.program_id(0)
    nb = pl.num_programs(0)
    ts, m = e1_ref.shape
    h = jnp.dot(ctx_ref[...], W1_ref[...],
                preferred_element_type=jnp.float32) + b1_ref[...]
    h = 0.5 * h * (1.0 + lax.erf(h * SQRT1_2))
    logits = jnp.dot(h, W2_ref[...],
                     preferred_element_type=jnp.float32) + b2_ref[...]
    mx = jnp.max(logits, axis=-1, keepdims=True)
    ex = jnp.exp(logits - mx)
    p = ex / jnp.sum(ex, axis=-1, keepdims=True)
    col = lax.broadcasted_iota(jnp.int32, p.shape, 1)
    big = jnp.int32(0x7FFFFFFF)
    m1 = jnp.max(p, axis=-1, keepdims=True)
    a1 = jnp.min(jnp.where(p == m1, col, big), axis=-1, keepdims=True)
    pm = jnp.where(col == a1, -1.0, p)
    m2 = jnp.max(pm, axis=-1, keepdims=True)
    a2 = jnp.min(jnp.where(pm == m2, col, big), axis=-1, keepdims=True)
    mask = (col == a1) | (col == a2)
    s = jnp.where(mask, p, 0.0)
    s = s / jnp.sum(s, axis=-1, keepdims=True)
    timing = jax.nn.sigmoid(
        jnp.dot(x_ref[...], WgT_ref[...],
                preferred_element_type=jnp.float32) + bg_ref[...])
    w = s * timing

    @pl.when(i == 0)
    def _init():
        carry_ref[...] = jnp.zeros_like(carry_ref)

    maskf = mask.astype(jnp.float32)
    rt = lax.broadcasted_iota(jnp.int32, (ts, ts), 0)
    ct = lax.broadcasted_iota(jnp.int32, (ts, ts), 1)
    strict_l = (rt > ct).astype(jnp.float32)
    rank = carry_ref[...] + jnp.dot(strict_l, maskf,
                                    preferred_element_type=jnp.float32)
    carry_ref[...] += jnp.sum(maskf, axis=0, keepdims=True)
    ranki = rank.astype(jnp.int32)

    e1_ref[...] = jnp.broadcast_to(a1, (ts, m))
    e2_ref[...] = jnp.broadcast_to(a2, (ts, m))
    r1_ref[...] = jnp.broadcast_to(
        jnp.sum(jnp.where(col == a1, ranki, 0), axis=-1, keepdims=True),
        (ts, m))
    r2_ref[...] = jnp.broadcast_to(
        jnp.sum(jnp.where(col == a2, ranki, 0), axis=-1, keepdims=True),
        (ts, m))
    w1_ref[...] = jnp.broadcast_to(
        jnp.sum(jnp.where(col == a1, w, 0.0), axis=-1, keepdims=True),
        (ts, m))
    w2_ref[...] = jnp.broadcast_to(
        jnp.sum(jnp.where(col == a2, w, 0.0), axis=-1, keepdims=True),
        (ts, m))

    @pl.when(i == nb - 1)
    def _fin():
        cnt = carry_ref[...]
        pb = jnp.ceil(cnt / TB)
        mi = lax.broadcasted_iota(jnp.int32, (m, m), 0)
        mj = lax.broadcasted_iota(jnp.int32, (m, m), 1)
        sl16 = (mi < mj).astype(jnp.float32)
        bb = jnp.dot(pb, sl16, preferred_element_type=jnp.float32)
        base_ref[...] = (bb * TB).astype(jnp.int32)
        eye = mi == mj
        bbc = jnp.sum(jnp.where(eye, jnp.broadcast_to(bb, (m, m)), 0.0),
                      axis=1, keepdims=True)
        nbw = be_ref.shape[1]
        bidx = lax.broadcasted_iota(jnp.int32, (m, nbw), 1)
        bbci = bbc.astype(jnp.int32)
        geq = (jnp.broadcast_to(bbci, (m, nbw)) <= bidx).astype(jnp.int32)
        be_ref[...] = jnp.sum(geq, axis=0, keepdims=True) - 1


def _sc_build_gather(x_hbm, e1_hbm, e2_hbm, r1_hbm, r2_hbm, w1_hbm, w2_hbm,
                     base_hbm, xg_hbm, roww_hbm, e1v, e2v, r1v, r2v, w1v, w2v,
                     basev, loc_tok, loc_w, *ring):
    rowsv = ring[:NBUF]
    sem = ring[NBUF:2 * NBUF]
    wsem = ring[2 * NBUF:3 * NBUF]
    s = e1v.shape[0]
    wid = lax.axis_index("s") * NC + lax.axis_index("c")
    rb = wid * RPW
    pltpu.sync_copy(e1_hbm, e1v)
    pltpu.sync_copy(e2_hbm, e2v)
    pltpu.sync_copy(r1_hbm, r1v)
    pltpu.sync_copy(r2_hbm, r2v)
    pltpu.sync_copy(w1_hbm, w1v)
    pltpu.sync_copy(w2_hbm, w2v)
    pltpu.sync_copy(base_hbm, basev)
    for t in range(RPW // L):
        loc_tok[pl.ds(t * L, L)] = jnp.zeros((L,), jnp.int32)
        loc_w[pl.ds(t * L, L)] = jnp.zeros((L,), jnp.float32)

    def scan_body(j, _):
        tok = j * L + lax.iota(jnp.int32, L)
        for ev, rv, wv in ((e1v, r1v, w1v), (e2v, r2v, w2v)):
            e = ev[pl.ds(j * L, L)]
            r = rv[pl.ds(j * L, L)]
            wgt = wv[pl.ds(j * L, L)]
            pos = plsc.load_gather(basev, [e]) + r
            msk = (pos >= rb) & (pos < rb + RPW)
            il = jnp.clip(pos - rb, 0, RPW - 1)
            plsc.store_scatter(loc_tok, [il], tok, mask=msk)
            plsc.store_scatter(loc_w, [il], wgt, mask=msk)
        return 0

    lax.fori_loop(0, s // L, scan_body, 0)
    pltpu.sync_copy(loc_w, roww_hbm.at[pl.ds(rb, RPW)])
    nsub = RPW // L
    gcp = [None] * nsub
    wcp = [None] * nsub
    for j in range(nsub):
        if j >= NOUT:
            k = j - NOUT
            gcp[k].wait()
            wcp[k] = pltpu.async_copy(
                rowsv[k % NBUF], xg_hbm.at[pl.ds(rb + k * L, L)],
                wsem[k % NBUF])
        if j >= NBUF:
            wcp[j - NBUF].wait()
        idxvec = loc_tok[pl.ds(j * L, L)]
        gcp[j] = pltpu.async_copy(x_hbm.at[idxvec], rowsv[j % NBUF],
                                  sem[j % NBUF])
    for k in range(nsub - NOUT, nsub):
        gcp[k].wait()
        wcp[k] = pltpu.async_copy(
            rowsv[k % NBUF], xg_hbm.at[pl.ds(rb + k * L, L)],
            wsem[k % NBUF])
    for k in range(nsub):
        if wcp[k] is not None and k >= nsub - NBUF:
            wcp[k].wait()


def _group_mm_body(be_ref, xg_ref, Wt_ref, bt_ref, ch_ref, w_ref, y_ref):
    y = jnp.dot(xg_ref[...].astype(jnp.bfloat16), Wt_ref[0],
                preferred_element_type=jnp.float32)
    y_ref[...] = (y + bt_ref[0] + ch_ref[0]) * w_ref[...]


def _sc_combine(y_hbm, e1_hbm, e2_hbm, r1_hbm, r2_hbm, base_hbm, sel_hbm,
                e1v, e2v, r1v, r2v, basev, p1v, p2v, p1c, p2c,
                buf_a, buf_b, sem):
    tpw = p1v.shape[0]
    d = buf_a.shape[1]
    wid = lax.axis_index("s") * NC + lax.axis_index("c")
    tb = wid * tpw
    pltpu.sync_copy(e1_hbm.at[pl.ds(tb, tpw)], e1v)
    pltpu.sync_copy(e2_hbm.at[pl.ds(tb, tpw)], e2v)
    pltpu.sync_copy(r1_hbm.at[pl.ds(tb, tpw)], r1v)
    pltpu.sync_copy(r2_hbm.at[pl.ds(tb, tpw)], r2v)
    pltpu.sync_copy(base_hbm, basev)
    for t in range(tpw // L):
        sl = pl.ds(t * L, L)
        p1v[sl] = plsc.load_gather(basev, [e1v[sl]]) + r1v[sl]
        p2v[sl] = plsc.load_gather(basev, [e2v[sl]]) + r2v[sl]
    for cchunk in range(tpw // CH):
        for t in range(CH // L):
            p1c[pl.ds(t * L, L)] = p1v[pl.ds(cchunk * CH + t * L, L)]
            p2c[pl.ds(t * L, L)] = p2v[pl.ds(cchunk * CH + t * L, L)]
        pltpu.async_copy(y_hbm.at[p1c], buf_a, sem).wait()
        pltpu.async_copy(y_hbm.at[p2c], buf_b, sem).wait()

        def add_row(rr, _):
            for cc in range(d // L):
                sl = pl.ds(cc * L, L)
                buf_a[rr, sl] = buf_a[rr, sl] + buf_b[rr, sl]
            return 0

        lax.fori_loop(0, CH, add_row, 0)
        pltpu.sync_copy(buf_a, sel_hbm.at[pl.ds(tb + cchunk * CH, CH)])


def _sc_mesh():
    return plsc.VectorSubcoreMesh(core_axis_name="c", subcore_axis_name="s",
                                  num_cores=NC, num_subcores=NS)


def _run_build_gather(xs, e1, e2, r1, r2, w1, w2, base16):
    S, D = xs.shape
    M = base16.shape[0]
    build = functools.partial(
        pl.kernel,
        out_type=[
            jax.ShapeDtypeStruct((NP, D), jnp.float32),
            jax.ShapeDtypeStruct((NP,), jnp.float32),
        ],
        mesh=_sc_mesh(),
        compiler_params=pltpu.CompilerParams(needs_layout_passes=False),
        scratch_types=[
            pltpu.VMEM((S,), jnp.int32),
            pltpu.VMEM((S,), jnp.int32),
            pltpu.VMEM((S,), jnp.int32),
            pltpu.VMEM((S,), jnp.int32),
            pltpu.VMEM((S,), jnp.float32),
            pltpu.VMEM((S,), jnp.float32),
            pltpu.VMEM((M,), jnp.int32),
            pltpu.VMEM((RPW,), jnp.int32),
            pltpu.VMEM((RPW,), jnp.float32),
        ] + [pltpu.VMEM((L, D), jnp.float32)] * NBUF
          + [pltpu.SemaphoreType.DMA] * (2 * NBUF),
    )(_sc_build_gather)
    return build(xs, e1, e2, r1, r2, w1, w2, base16)


def _run_combine(y, e1, e2, r1, r2, base16):
    S = e1.shape[0]
    D = y.shape[1]
    M = base16.shape[0]
    tpw = S // NW
    combine = functools.partial(
        pl.kernel,
        out_type=jax.ShapeDtypeStruct((S, D), jnp.float32),
        mesh=_sc_mesh(),
        compiler_params=pltpu.CompilerParams(needs_layout_passes=False),
        scratch_types=[
            pltpu.VMEM((tpw,), jnp.int32),
            pltpu.VMEM((tpw,), jnp.int32),
            pltpu.VMEM((tpw,), jnp.int32),
            pltpu.VMEM((tpw,), jnp.int32),
            pltpu.VMEM((M,), jnp.int32),
            pltpu.VMEM((tpw,), jnp.int32),
            pltpu.VMEM((tpw,), jnp.int32),
            pltpu.VMEM((CH,), jnp.int32),
            pltpu.VMEM((CH,), jnp.int32),
            pltpu.VMEM((CH, D), jnp.float32),
            pltpu.VMEM((CH, D), jnp.float32),
            pltpu.SemaphoreType.DMA,
        ],
    )(_sc_combine)
    return combine(y, e1, e2, r1, r2, base16)


def _integrate_body(x_ref, sel_ref, Wi_ref, bi_ref, out_ref):
    d = x_ref.shape[-1]
    out_ref[...] = (
        jnp.dot(x_ref[...], Wi_ref[:d, :], preferred_element_type=jnp.float32)
        + jnp.dot(sel_ref[...], Wi_ref[d:, :],
                  preferred_element_type=jnp.float32)
        + bi_ref[...])


def kernel(x, context, Wt, bt, Wg, bg, ch, W1, b1, W2, b2, Wi, bi):
    B, S, D = x.shape
    M = Wt.shape[0]
    H = W1.shape[1]
    xs = x.reshape(S, D)
    cs = context.reshape(S, D)
    TS = 256
    n_blk = S // TS

    outs = pl.pallas_call(
        _route_body,
        grid=(n_blk,),
        in_specs=[
            pl.BlockSpec((TS, D), lambda i: (i, 0)),
            pl.BlockSpec((TS, D), lambda i: (i, 0)),
            pl.BlockSpec((D, H), lambda i: (0, 0)),
            pl.BlockSpec((1, H), lambda i: (0, 0)),
            pl.BlockSpec((H, M), lambda i: (0, 0)),
            pl.BlockSpec((1, M), lambda i: (0, 0)),
            pl.BlockSpec((D, M), lambda i: (0, 0)),
            pl.BlockSpec((1, M), lambda i: (0, 0)),
        ],
        out_specs=[
            pl.BlockSpec((TS, M), lambda i: (i, 0)),
            pl.BlockSpec((TS, M), lambda i: (i, 0)),
            pl.BlockSpec((TS, M), lambda i: (i, 0)),
            pl.BlockSpec((TS, M), lambda i: (i, 0)),
            pl.BlockSpec((TS, M), lambda i: (i, 0)),
            pl.BlockSpec((TS, M), lambda i: (i, 0)),
            pl.BlockSpec((1, M), lambda i: (0, 0)),
            pl.BlockSpec((1, 64), lambda i: (0, 0)),
        ],
        out_shape=[
            jax.ShapeDtypeStruct((S, M), jnp.int32),
            jax.ShapeDtypeStruct((S, M), jnp.int32),
            jax.ShapeDtypeStruct((S, M), jnp.int32),
            jax.ShapeDtypeStruct((S, M), jnp.int32),
            jax.ShapeDtypeStruct((S, M), jnp.float32),
            jax.ShapeDtypeStruct((S, M), jnp.float32),
            jax.ShapeDtypeStruct((1, M), jnp.int32),
            jax.ShapeDtypeStruct((1, 64), jnp.int32),
        ],
        scratch_shapes=[pltpu.VMEM((1, M), jnp.float32)],
        compiler_params=pltpu.CompilerParams(
            dimension_semantics=("arbitrary",)),
    )(cs, xs, W1, b1.reshape(1, H), W2, b2.reshape(1, M), Wg.T,
      bg.reshape(1, M))
    e1b, e2b, r1b, r2b, w1b, w2b, base, beb = outs
    e1, e2, r1, r2 = e1b[:, 0], e2b[:, 0], r1b[:, 0], r2b[:, 0]
    w1, w2 = w1b[:, 0], w2b[:, 0]
    base16 = base.reshape(M)
    be = beb.reshape(64)[:NBK]

    xg, roww = _run_build_gather(xs, e1, e2, r1, r2, w1, w2, base16)
    y = pl.pallas_call(
        _group_mm_body,
        grid_spec=pltpu.PrefetchScalarGridSpec(
            num_scalar_prefetch=1,
            grid=(NBK,),
            in_specs=[
                pl.BlockSpec((TB, D), lambda i, be: (i, 0)),
                pl.BlockSpec((1, D, D), lambda i, be: (be[i], 0, 0)),
                pl.BlockSpec((1, 1, D), lambda i, be: (be[i], 0, 0)),
                pl.BlockSpec((1, 1, D), lambda i, be: (be[i], 0, 0)),
                pl.BlockSpec((TB, 1), lambda i, be: (i, 0)),
            ],
            out_specs=pl.BlockSpec((TB, D), lambda i, be: (i, 0)),
        ),
        out_shape=jax.ShapeDtypeStruct((NP, D), jnp.float32),
        compiler_params=pltpu.CompilerParams(
            dimension_semantics=("arbitrary",)),
    )(be, xg, Wt.astype(jnp.bfloat16), bt.reshape(M, 1, D), ch.reshape(M, 1, D), roww.reshape(NP, 1))

    sel = _run_combine(y, e1, e2, r1, r2, base16)

    out = pl.pallas_call(
        _integrate_body,
        grid=(n_blk,),
        in_specs=[
            pl.BlockSpec((TS, D), lambda i: (i, 0)),
            pl.BlockSpec((TS, D), lambda i: (i, 0)),
            pl.BlockSpec((2 * D, D), lambda i: (0, 0)),
            pl.BlockSpec((1, D), lambda i: (0, 0)),
        ],
        out_specs=pl.BlockSpec((TS, D), lambda i: (i, 0)),
        out_shape=jax.ShapeDtypeStruct((S, D), jnp.float32),
    )(xs, sel, Wi, bi.reshape(1, D))

    return out.reshape(B, S, D)


# speedup vs baseline: 1.5075x; 1.5024x over previous
import functools

import jax
import jax.numpy as jnp
from jax import lax
from jax.experimental import pallas as pl
from jax.experimental.pallas import tpu as pltpu
from jax.experimental.pallas import tpu_sc as plsc

NC, NS, L = 2, 16, 16
NW = NC * NS
TB = 128
NBK = 48
NP = NBK * TB
RPW = NP // NW
GC = 48
NBUF = 7
NOUT = 6
CH = 32
SQRT1_2 = 0.7071067811865476


def _route_body(ctx_ref, x_ref, W1_ref, b1_ref, W2_ref, b2_ref, WgT_ref,
                bg_ref, e1_ref, e2_ref, r1_ref, r2_ref, w1_ref, w2_ref,
                base_ref, be_ref, carry_ref):
    i = pl.program_id(0)
    nb = pl.num_programs(0)
    ts, m = e1_ref.shape
    h = jnp.dot(ctx_ref[...], W1_ref[...],
                preferred_element_type=jnp.float32) + b1_ref[...]
    h = 0.5 * h * (1.0 + lax.erf(h * SQRT1_2))
    logits = jnp.dot(h, W2_ref[...],
                     preferred_element_type=jnp.float32) + b2_ref[...]
    mx = jnp.max(logits, axis=-1, keepdims=True)
    ex = jnp.exp(logits - mx)
    p = ex / jnp.sum(ex, axis=-1, keepdims=True)
    col = lax.broadcasted_iota(jnp.int32, p.shape, 1)
    big = jnp.int32(0x7FFFFFFF)
    m1 = jnp.max(p, axis=-1, keepdims=True)
    a1 = jnp.min(jnp.where(p == m1, col, big), axis=-1, keepdims=True)
    pm = jnp.where(col == a1, -1.0, p)
    m2 = jnp.max(pm, axis=-1, keepdims=True)
    a2 = jnp.min(jnp.where(pm == m2, col, big), axis=-1, keepdims=True)
    mask = (col == a1) | (col == a2)
    s = jnp.where(mask, p, 0.0)
    s = s / jnp.sum(s, axis=-1, keepdims=True)
    timing = jax.nn.sigmoid(
        jnp.dot(x_ref[...], WgT_ref[...],
                preferred_element_type=jnp.float32) + bg_ref[...])
    w = s * timing

    @pl.when(i == 0)
    def _init():
        carry_ref[...] = jnp.zeros_like(carry_ref)

    maskf = mask.astype(jnp.float32)
    rt = lax.broadcasted_iota(jnp.int32, (ts, ts), 0)
    ct = lax.broadcasted_iota(jnp.int32, (ts, ts), 1)
    strict_l = (rt > ct).astype(jnp.float32)
    rank = carry_ref[...] + jnp.dot(strict_l, maskf,
                                    preferred_element_type=jnp.float32)
    carry_ref[...] += jnp.sum(maskf, axis=0, keepdims=True)
    ranki = rank.astype(jnp.int32)

    e1_ref[...] = jnp.broadcast_to(a1, (ts, m))
    e2_ref[...] = jnp.broadcast_to(a2, (ts, m))
    r1_ref[...] = jnp.broadcast_to(
        jnp.sum(jnp.where(col == a1, ranki, 0), axis=-1, keepdims=True),
        (ts, m))
    r2_ref[...] = jnp.broadcast_to(
        jnp.sum(jnp.where(col == a2, ranki, 0), axis=-1, keepdims=True),
        (ts, m))
    w1_ref[...] = jnp.broadcast_to(
        jnp.sum(jnp.where(col == a1, w, 0.0), axis=-1, keepdims=True),
        (ts, m))
    w2_ref[...] = jnp.broadcast_to(
        jnp.sum(jnp.where(col == a2, w, 0.0), axis=-1, keepdims=True),
        (ts, m))

    @pl.when(i == nb - 1)
    def _fin():
        cnt = carry_ref[...]
        pb = jnp.ceil(cnt / TB)
        mi = lax.broadcasted_iota(jnp.int32, (m, m), 0)
        mj = lax.broadcasted_iota(jnp.int32, (m, m), 1)
        sl16 = (mi < mj).astype(jnp.float32)
        bb = jnp.dot(pb, sl16, preferred_element_type=jnp.float32)
        base_ref[...] = (bb * TB).astype(jnp.int32)
        eye = mi == mj
        bbc = jnp.sum(jnp.where(eye, jnp.broadcast_to(bb, (m, m)), 0.0),
                      axis=1, keepdims=True)
        nbw = be_ref.shape[1]
        bidx = lax.broadcasted_iota(jnp.int32, (m, nbw), 1)
        bbci = bbc.astype(jnp.int32)
        geq = (jnp.broadcast_to(bbci, (m, nbw)) <= bidx).astype(jnp.int32)
        be_ref[...] = jnp.sum(geq, axis=0, keepdims=True) - 1


def _sc_build_gather(x_hbm, e1_hbm, e2_hbm, r1_hbm, r2_hbm, w1_hbm, w2_hbm,
                     base_hbm, xg_hbm, roww_hbm, e1v, e2v, r1v, r2v, w1v, w2v,
                     basev, loc_tok, loc_w, *ring):
    rowsv = ring[:NBUF]
    sem = ring[NBUF:2 * NBUF]
    wsem = ring[2 * NBUF:3 * NBUF]
    s = e1v.shape[0]
    wid = lax.axis_index("s") * NC + lax.axis_index("c")
    rb = wid * RPW
    pltpu.sync_copy(e1_hbm, e1v)
    pltpu.sync_copy(e2_hbm, e2v)
    pltpu.sync_copy(r1_hbm, r1v)
    pltpu.sync_copy(r2_hbm, r2v)
    pltpu.sync_copy(w1_hbm, w1v)
    pltpu.sync_copy(w2_hbm, w2v)
    pltpu.sync_copy(base_hbm, basev)
    for t in range(RPW // L):
        spread = jnp.bitwise_and(rb + t * L + lax.iota(jnp.int32, L),
                                 s - 1)
        loc_tok[pl.ds(t * L, L)] = spread
        loc_w[pl.ds(t * L, L)] = jnp.zeros((L,), jnp.float32)

    def scan_body(j, _):
        tok = j * L + lax.iota(jnp.int32, L)
        for ev, rv, wv in ((e1v, r1v, w1v), (e2v, r2v, w2v)):
            e = ev[pl.ds(j * L, L)]
            r = rv[pl.ds(j * L, L)]
            wgt = wv[pl.ds(j * L, L)]
            pos = plsc.load_gather(basev, [e]) + r
            msk = (pos >= rb) & (pos < rb + RPW)
            il = jnp.clip(pos - rb, 0, RPW - 1)
            plsc.store_scatter(loc_tok, [il], tok, mask=msk)
            plsc.store_scatter(loc_w, [il], wgt, mask=msk)
        return 0

    lax.fori_loop(0, s // L, scan_body, 0)
    pltpu.sync_copy(loc_w, roww_hbm.at[pl.ds(rb, RPW)])
    nsub = RPW // L
    gcp = [None] * nsub
    wcp = [None] * nsub
    for j in range(nsub):
        if j >= NOUT:
            k = j - NOUT
            gcp[k].wait()
            wcp[k] = pltpu.async_copy(
                rowsv[k % NBUF], xg_hbm.at[pl.ds(rb + k * L, L)],
                wsem[k % NBUF])
        if j >= NBUF:
            wcp[j - NBUF].wait()
        idxvec = loc_tok[pl.ds(j * L, L)]
        gcp[j] = pltpu.async_copy(x_hbm.at[idxvec], rowsv[j % NBUF],
                                  sem[j % NBUF])
    for k in range(nsub - NOUT, nsub):
        gcp[k].wait()
        wcp[k] = pltpu.async_copy(
            rowsv[k % NBUF], xg_hbm.at[pl.ds(rb + k * L, L)],
            wsem[k % NBUF])
    for k in range(nsub):
        if wcp[k] is not None and k >= nsub - NBUF:
            wcp[k].wait()


def _group_mm_body(be_ref, xg_ref, Wt_ref, bt_ref, ch_ref, w_ref, y_ref):
    y = jnp.dot(xg_ref[...].astype(jnp.bfloat16), Wt_ref[0],
                preferred_element_type=jnp.float32)
    y_ref[...] = (y + bt_ref[0] + ch_ref[0]) * w_ref[...]


def _sc_combine(y_hbm, e1_hbm, e2_hbm, r1_hbm, r2_hbm, base_hbm, sel_hbm,
                e1v, e2v, r1v, r2v, basev, p1v, p2v, p1c, p2c,
                buf_a, buf_b, sem):
    tpw = p1v.shape[0]
    d = buf_a.shape[1]
    wid = lax.axis_index("s") * NC + lax.axis_index("c")
    tb = wid * tpw
    pltpu.sync_copy(e1_hbm.at[pl.ds(tb, tpw)], e1v)
    pltpu.sync_copy(e2_hbm.at[pl.ds(tb, tpw)], e2v)
    pltpu.sync_copy(r1_hbm.at[pl.ds(tb, tpw)], r1v)
    pltpu.sync_copy(r2_hbm.at[pl.ds(tb, tpw)], r2v)
    pltpu.sync_copy(base_hbm, basev)
    for t in range(tpw // L):
        sl = pl.ds(t * L, L)
        p1v[sl] = plsc.load_gather(basev, [e1v[sl]]) + r1v[sl]
        p2v[sl] = plsc.load_gather(basev, [e2v[sl]]) + r2v[sl]
    for cchunk in range(tpw // CH):
        for t in range(CH // L):
            p1c[pl.ds(t * L, L)] = p1v[pl.ds(cchunk * CH + t * L, L)]
            p2c[pl.ds(t * L, L)] = p2v[pl.ds(cchunk * CH + t * L, L)]
        pltpu.async_copy(y_hbm.at[p1c], buf_a, sem).wait()
        pltpu.async_copy(y_hbm.at[p2c], buf_b, sem).wait()

        def add_row(rr, _):
            for cc in range(d // L):
                sl = pl.ds(cc * L, L)
                buf_a[rr, sl] = buf_a[rr, sl] + buf_b[rr, sl]
            return 0

        lax.fori_loop(0, CH, add_row, 0)
        pltpu.sync_copy(buf_a, sel_hbm.at[pl.ds(tb + cchunk * CH, CH)])


def _sc_mesh():
    return plsc.VectorSubcoreMesh(core_axis_name="c", subcore_axis_name="s",
                                  num_cores=NC, num_subcores=NS)


def _run_build_gather(xs, e1, e2, r1, r2, w1, w2, base16):
    S, D = xs.shape
    M = base16.shape[0]
    build = functools.partial(
        pl.kernel,
        out_type=[
            jax.ShapeDtypeStruct((NP, D), jnp.float32),
            jax.ShapeDtypeStruct((NP,), jnp.float32),
        ],
        mesh=_sc_mesh(),
        compiler_params=pltpu.CompilerParams(needs_layout_passes=False),
        scratch_types=[
            pltpu.VMEM((S,), jnp.int32),
            pltpu.VMEM((S,), jnp.int32),
            pltpu.VMEM((S,), jnp.int32),
            pltpu.VMEM((S,), jnp.int32),
            pltpu.VMEM((S,), jnp.float32),
            pltpu.VMEM((S,), jnp.float32),
            pltpu.VMEM((M,), jnp.int32),
            pltpu.VMEM((RPW,), jnp.int32),
            pltpu.VMEM((RPW,), jnp.float32),
        ] + [pltpu.VMEM((L, D), jnp.float32)] * NBUF
          + [pltpu.SemaphoreType.DMA] * (2 * NBUF),
    )(_sc_build_gather)
    return build(xs, e1, e2, r1, r2, w1, w2, base16)


def _run_combine(y, e1, e2, r1, r2, base16):
    S = e1.shape[0]
    D = y.shape[1]
    M = base16.shape[0]
    tpw = S // NW
    combine = functools.partial(
        pl.kernel,
        out_type=jax.ShapeDtypeStruct((S, D), jnp.float32),
        mesh=_sc_mesh(),
        compiler_params=pltpu.CompilerParams(needs_layout_passes=False),
        scratch_types=[
            pltpu.VMEM((tpw,), jnp.int32),
            pltpu.VMEM((tpw,), jnp.int32),
            pltpu.VMEM((tpw,), jnp.int32),
            pltpu.VMEM((tpw,), jnp.int32),
            pltpu.VMEM((M,), jnp.int32),
            pltpu.VMEM((tpw,), jnp.int32),
            pltpu.VMEM((tpw,), jnp.int32),
            pltpu.VMEM((CH,), jnp.int32),
            pltpu.VMEM((CH,), jnp.int32),
            pltpu.VMEM((CH, D), jnp.float32),
            pltpu.VMEM((CH, D), jnp.float32),
            pltpu.SemaphoreType.DMA,
        ],
    )(_sc_combine)
    return combine(y, e1, e2, r1, r2, base16)


def _integrate_body(x_ref, sel_ref, Wi_ref, bi_ref, out_ref):
    d = x_ref.shape[-1]
    out_ref[...] = (
        jnp.dot(x_ref[...], Wi_ref[:d, :], preferred_element_type=jnp.float32)
        + jnp.dot(sel_ref[...], Wi_ref[d:, :],
                  preferred_element_type=jnp.float32)
        + bi_ref[...])


def kernel(x, context, Wt, bt, Wg, bg, ch, W1, b1, W2, b2, Wi, bi):
    B, S, D = x.shape
    M = Wt.shape[0]
    H = W1.shape[1]
    xs = x.reshape(S, D)
    cs = context.reshape(S, D)
    TS = 256
    n_blk = S // TS

    outs = pl.pallas_call(
        _route_body,
        grid=(n_blk,),
        in_specs=[
            pl.BlockSpec((TS, D), lambda i: (i, 0)),
            pl.BlockSpec((TS, D), lambda i: (i, 0)),
            pl.BlockSpec((D, H), lambda i: (0, 0)),
            pl.BlockSpec((1, H), lambda i: (0, 0)),
            pl.BlockSpec((H, M), lambda i: (0, 0)),
            pl.BlockSpec((1, M), lambda i: (0, 0)),
            pl.BlockSpec((D, M), lambda i: (0, 0)),
            pl.BlockSpec((1, M), lambda i: (0, 0)),
        ],
        out_specs=[
            pl.BlockSpec((TS, M), lambda i: (i, 0)),
            pl.BlockSpec((TS, M), lambda i: (i, 0)),
            pl.BlockSpec((TS, M), lambda i: (i, 0)),
            pl.BlockSpec((TS, M), lambda i: (i, 0)),
            pl.BlockSpec((TS, M), lambda i: (i, 0)),
            pl.BlockSpec((TS, M), lambda i: (i, 0)),
            pl.BlockSpec((1, M), lambda i: (0, 0)),
            pl.BlockSpec((1, 64), lambda i: (0, 0)),
        ],
        out_shape=[
            jax.ShapeDtypeStruct((S, M), jnp.int32),
            jax.ShapeDtypeStruct((S, M), jnp.int32),
            jax.ShapeDtypeStruct((S, M), jnp.int32),
            jax.ShapeDtypeStruct((S, M), jnp.int32),
            jax.ShapeDtypeStruct((S, M), jnp.float32),
            jax.ShapeDtypeStruct((S, M), jnp.float32),
            jax.ShapeDtypeStruct((1, M), jnp.int32),
            jax.ShapeDtypeStruct((1, 64), jnp.int32),
        ],
        scratch_shapes=[pltpu.VMEM((1, M), jnp.float32)],
        compiler_params=pltpu.CompilerParams(
            dimension_semantics=("arbitrary",)),
    )(cs, xs, W1, b1.reshape(1, H), W2, b2.reshape(1, M), Wg.T,
      bg.reshape(1, M))
    e1b, e2b, r1b, r2b, w1b, w2b, base, beb = outs
    e1, e2, r1, r2 = e1b[:, 0], e2b[:, 0], r1b[:, 0], r2b[:, 0]
    w1, w2 = w1b[:, 0], w2b[:, 0]
    base16 = base.reshape(M)
    be = beb.reshape(64)[:NBK]

    xg, roww = _run_build_gather(xs, e1, e2, r1, r2, w1, w2, base16)
    y = pl.pallas_call(
        _group_mm_body,
        grid_spec=pltpu.PrefetchScalarGridSpec(
            num_scalar_prefetch=1,
            grid=(NBK,),
            in_specs=[
                pl.BlockSpec((TB, D), lambda i, be: (i, 0)),
                pl.BlockSpec((1, D, D), lambda i, be: (be[i], 0, 0)),
                pl.BlockSpec((1, 1, D), lambda i, be: (be[i], 0, 0)),
                pl.BlockSpec((1, 1, D), lambda i, be: (be[i], 0, 0)),
                pl.BlockSpec((TB, 1), lambda i, be: (i, 0)),
            ],
            out_specs=pl.BlockSpec((TB, D), lambda i, be: (i, 0)),
        ),
        out_shape=jax.ShapeDtypeStruct((NP, D), jnp.float32),
        compiler_params=pltpu.CompilerParams(
            dimension_semantics=("arbitrary",)),
    )(be, xg, Wt.astype(jnp.bfloat16), bt.reshape(M, 1, D), ch.reshape(M, 1, D), roww.reshape(NP, 1))

    sel = _run_combine(y, e1, e2, r1, r2, base16)

    out = pl.pallas_call(
        _integrate_body,
        grid=(n_blk,),
        in_specs=[
            pl.BlockSpec((TS, D), lambda i: (i, 0)),
            pl.BlockSpec((TS, D), lambda i: (i, 0)),
            pl.BlockSpec((2 * D, D), lambda i: (0, 0)),
            pl.BlockSpec((1, D), lambda i: (0, 0)),
        ],
        out_specs=pl.BlockSpec((TS, D), lambda i: (i, 0)),
        out_shape=jax.ShapeDtypeStruct((S, D), jnp.float32),
    )(xs, sel, Wi, bi.reshape(1, D))

    return out.reshape(B, S, D)
